# baseline (speedup 1.0000x reference)
import functools

import jax
import jax.numpy as jnp
from jax.experimental import pallas as pl
from jax.experimental.pallas import tpu as pltpu

HIDDEN = 2048
NUM_EXPERTS = 64
TOP_K = 8
TOKENS = 16384

BLK = 512


def _router_block(x_ref, w_ref, ids_ref, wts_ref, logits_ref):
    x = x_ref[...]
    w = w_ref[...]
    logits = jax.lax.dot_general(
        x, w, dimension_numbers=(((1,), (1,)), ((), ())),
        preferred_element_type=jnp.float32)
    logits_ref[...] = logits

    col = jax.lax.broadcasted_iota(jnp.int32, (BLK, NUM_EXPERTS), 1)
    work = logits
    vals = []
    idxs = []
    for _ in range(TOP_K):
        m = jnp.max(work, axis=1, keepdims=True)
        idx = jnp.min(jnp.where(work == m, col, NUM_EXPERTS), axis=1,
                      keepdims=True)
        vals.append(m)
        idxs.append(idx)
        work = jnp.where(col == idx, -jnp.inf, work)
    topv = jnp.concatenate(vals, axis=1)
    topi = jnp.concatenate(idxs, axis=1)
    e = jnp.exp(topv - topv[:, 0:1])
    wts_ref[...] = e / jnp.sum(e, axis=1, keepdims=True)
    ids_ref[...] = topi


@jax.jit
def kernel(hidden_states, gate_weight):
    grid = (TOKENS // BLK,)
    ids, wts, logits = pl.pallas_call(
        _router_block,
        grid=grid,
        in_specs=[
            pl.BlockSpec((BLK, HIDDEN), lambda i: (i, 0)),
            pl.BlockSpec((NUM_EXPERTS, HIDDEN), lambda i: (0, 0)),
        ],
        out_specs=[
            pl.BlockSpec((BLK, TOP_K), lambda i: (i, 0)),
            pl.BlockSpec((BLK, TOP_K), lambda i: (i, 0)),
            pl.BlockSpec((BLK, NUM_EXPERTS), lambda i: (i, 0)),
        ],
        out_shape=[
            jax.ShapeDtypeStruct((TOKENS, TOP_K), jnp.int32),
            jax.ShapeDtypeStruct((TOKENS, TOP_K), jnp.float32),
            jax.ShapeDtypeStruct((TOKENS, NUM_EXPERTS), jnp.float32),
        ],
        compiler_params=pltpu.CompilerParams(
            dimension_semantics=("arbitrary",)),
    )(hidden_states, gate_weight)
    return ids, wts, logits


# speedup vs baseline: 1.0310x; 1.0310x over previous
import functools

import jax
import jax.numpy as jnp
from jax import lax
from jax.experimental import pallas as pl
from jax.experimental.pallas import tpu as pltpu
from jax.experimental.pallas import tpu_sc as plsc

HIDDEN = 2048
NUM_EXPERTS = 64
TOP_K = 8
TOKENS = 16384

MM_BLK = 512
NC, NS, L = 2, 16, 16
NW = NC * NS
R_PER_W = TOKENS // NW


def _matmul_block(x_ref, w_ref, logits_ref):
    logits_ref[...] = jax.lax.dot_general(
        x_ref[...], w_ref[...], dimension_numbers=(((1,), (1,)), ((), ())),
        preferred_element_type=jnp.float32)


def _tc_logits(hidden_states, gate_weight):
    return pl.pallas_call(
        _matmul_block,
        grid=(TOKENS // MM_BLK,),
        in_specs=[
            pl.BlockSpec((MM_BLK, HIDDEN), lambda i: (i, 0)),
            pl.BlockSpec((NUM_EXPERTS, HIDDEN), lambda i: (0, 0)),
        ],
        out_specs=pl.BlockSpec((MM_BLK, NUM_EXPERTS), lambda i: (i, 0)),
        out_shape=jax.ShapeDtypeStruct((TOKENS, NUM_EXPERTS), jnp.float32),
        compiler_params=pltpu.CompilerParams(
            dimension_semantics=("arbitrary",)),
    )(hidden_states, gate_weight)


def _merge_desc(ak, av, bk, bv):
    rbk = lax.rev(bk, (0,))
    rbv = lax.rev(bv, (0,))
    take_a = (ak > rbk) | ((ak == rbk) & (av < rbv))
    mk = jnp.where(take_a, ak, rbk)
    mv = jnp.where(take_a, av, rbv)
    return plsc.sort_key_val(mk, mv, descending=True)


def _sc_route(logits):
    mesh = plsc.VectorSubcoreMesh(core_axis_name="c", subcore_axis_name="s")

    @functools.partial(
        pl.kernel,
        mesh=mesh,
        out_type=[
            jax.ShapeDtypeStruct((TOKENS * TOP_K,), jnp.int32),
            jax.ShapeDtypeStruct((TOKENS * TOP_K,), jnp.float32),
        ],
        scratch_types=[
            pltpu.VMEM((R_PER_W, NUM_EXPERTS), jnp.float32),
            pltpu.VMEM((R_PER_W * TOP_K,), jnp.int32),
            pltpu.VMEM((R_PER_W * TOP_K,), jnp.float32),
        ],
        compiler_params=pltpu.CompilerParams(needs_layout_passes=False),
    )
    def sc_topk(logits_hbm, ids_hbm, wts_hbm, lg_v, ids_v, wts_v):
        wid = lax.axis_index("s") * NC + lax.axis_index("c")
        base = wid * R_PER_W
        pltpu.sync_copy(logits_hbm.at[pl.ds(base, R_PER_W)], lg_v)

        lane = lax.iota(jnp.int32, L)
        lane8 = lane < TOP_K

        def row_body(r, carry):
            sorted_kv = []
            for c in range(NUM_EXPERTS // L):
                k = lg_v[r, pl.ds(c * L, L)]
                sorted_kv.append(
                    plsc.sort_key_val(k, lane + c * L, descending=True))
            t01 = _merge_desc(*sorted_kv[0], *sorted_kv[1])
            t23 = _merge_desc(*sorted_kv[2], *sorted_kv[3])
            fk, fv = _merge_desc(*t01, *t23)
            e = jnp.exp(fk - jnp.max(fk))
            e8 = jnp.where(lane8, e, 0.0)
            w = e8 / jnp.sum(e8)
            pos = r * TOP_K + lane
            plsc.store_scatter(ids_v, [pos], fv, mask=lane8)
            plsc.store_scatter(wts_v, [pos], w, mask=lane8)
            return carry

        lax.fori_loop(0, R_PER_W, row_body, 0)
        pltpu.sync_copy(ids_v, ids_hbm.at[pl.ds(base * TOP_K, R_PER_W * TOP_K)])
        pltpu.sync_copy(wts_v, wts_hbm.at[pl.ds(base * TOP_K, R_PER_W * TOP_K)])

    ids_flat, wts_flat = sc_topk(logits)
    return (ids_flat.reshape(TOKENS, TOP_K), wts_flat.reshape(TOKENS, TOP_K))


@jax.jit
def kernel(hidden_states, gate_weight):
    logits = _tc_logits(hidden_states, gate_weight)
    ids, wts = _sc_route(logits)
    return ids, wts, logits


# speedup vs baseline: 1.2760x; 1.2376x over previous
import functools

import jax
import jax.numpy as jnp
from jax import lax
from jax.experimental import pallas as pl
from jax.experimental.pallas import tpu as pltpu
from jax.experimental.pallas import tpu_sc as plsc

HIDDEN = 2048
NUM_EXPERTS = 64
TOP_K = 8
TOKENS = 16384

MM_BLK = 512
NC, NS, L = 2, 16, 16
NW = NC * NS
R_PER_W = TOKENS // NW


def _matmul_block(x_ref, w_ref, logits_ref):
    logits_ref[...] = jax.lax.dot_general(
        x_ref[...], w_ref[...], dimension_numbers=(((1,), (1,)), ((), ())),
        preferred_element_type=jnp.float32)


def _tc_logits(hidden_states, gate_weight):
    return pl.pallas_call(
        _matmul_block,
        grid=(TOKENS // MM_BLK,),
        in_specs=[
            pl.BlockSpec((MM_BLK, HIDDEN), lambda i: (i, 0)),
            pl.BlockSpec((NUM_EXPERTS, HIDDEN), lambda i: (0, 0)),
        ],
        out_specs=pl.BlockSpec((MM_BLK, NUM_EXPERTS), lambda i: (i, 0)),
        out_shape=jax.ShapeDtypeStruct((TOKENS, NUM_EXPERTS), jnp.float32),
        compiler_params=pltpu.CompilerParams(
            dimension_semantics=("arbitrary",)),
    )(hidden_states, gate_weight)


def _merge_desc(ak, av, bk, bv):
    rbk = lax.rev(bk, (0,))
    rbv = lax.rev(bv, (0,))
    take_a = (ak > rbk) | ((ak == rbk) & (av < rbv))
    mk = jnp.where(take_a, ak, rbk)
    mv = jnp.where(take_a, av, rbv)
    return plsc.sort_key_val(mk, mv, descending=True)


def _sc_route(logits):
    mesh = plsc.VectorSubcoreMesh(core_axis_name="c", subcore_axis_name="s")

    @functools.partial(
        pl.kernel,
        mesh=mesh,
        out_type=[
            jax.ShapeDtypeStruct((TOKENS * TOP_K,), jnp.int32),
            jax.ShapeDtypeStruct((TOKENS * TOP_K,), jnp.float32),
        ],
        scratch_types=[
            pltpu.VMEM((R_PER_W, NUM_EXPERTS), jnp.float32),
            pltpu.VMEM((R_PER_W * TOP_K,), jnp.int32),
            pltpu.VMEM((R_PER_W * TOP_K,), jnp.float32),
        ],
        compiler_params=pltpu.CompilerParams(needs_layout_passes=False),
    )
    def sc_topk(logits_hbm, ids_hbm, wts_hbm, lg_v, ids_v, wts_v):
        wid = lax.axis_index("s") * NC + lax.axis_index("c")
        base = wid * R_PER_W
        pltpu.sync_copy(logits_hbm.at[pl.ds(base, R_PER_W)], lg_v)

        lane = lax.iota(jnp.int32, L)
        lane8 = lane < TOP_K

        @plsc.parallel_loop(0, R_PER_W, unroll=8)
        def row_body(r):
            sorted_kv = []
            for c in range(NUM_EXPERTS // L):
                k = lg_v[r, pl.ds(c * L, L)]
                sorted_kv.append(
                    plsc.sort_key_val(k, lane + c * L, descending=True))
            t01 = _merge_desc(*sorted_kv[0], *sorted_kv[1])
            t23 = _merge_desc(*sorted_kv[2], *sorted_kv[3])
            fk, fv = _merge_desc(*t01, *t23)
            e = jnp.exp(fk - jnp.max(fk))
            e8 = jnp.where(lane8, e, 0.0)
            w = e8 / jnp.sum(e8)
            pos = r * TOP_K + lane
            plsc.store_scatter(ids_v, [pos], fv, mask=lane8)
            plsc.store_scatter(wts_v, [pos], w, mask=lane8)
        pltpu.sync_copy(ids_v, ids_hbm.at[pl.ds(base * TOP_K, R_PER_W * TOP_K)])
        pltpu.sync_copy(wts_v, wts_hbm.at[pl.ds(base * TOP_K, R_PER_W * TOP_K)])

    ids_flat, wts_flat = sc_topk(logits)
    return (ids_flat.reshape(TOKENS, TOP_K), wts_flat.reshape(TOKENS, TOP_K))


@jax.jit
def kernel(hidden_states, gate_weight):
    logits = _tc_logits(hidden_states, gate_weight)
    ids, wts = _sc_route(logits)
    return ids, wts, logits


# speedup vs baseline: 1.3743x; 1.0771x over previous
import functools

import jax
import jax.numpy as jnp
from jax import lax
from jax.experimental import pallas as pl
from jax.experimental.pallas import tpu as pltpu
from jax.experimental.pallas import tpu_sc as plsc

HIDDEN = 2048
NUM_EXPERTS = 64
TOP_K = 8
TOKENS = 16384

MM_BLK = 1024
NC, NS, L = 2, 16, 16
NW = NC * NS
R_PER_W = TOKENS // NW


def _matmul_block(x_ref, w_ref, logits_ref):
    logits_ref[...] = jax.lax.dot_general(
        x_ref[...], w_ref[...], dimension_numbers=(((1,), (1,)), ((), ())),
        preferred_element_type=jnp.float32)


def _tc_logits(hidden_states, gate_weight):
    return pl.pallas_call(
        _matmul_block,
        grid=(TOKENS // MM_BLK,),
        in_specs=[
            pl.BlockSpec((MM_BLK, HIDDEN), lambda i: (i, 0)),
            pl.BlockSpec((NUM_EXPERTS, HIDDEN), lambda i: (0, 0)),
        ],
        out_specs=pl.BlockSpec((MM_BLK, NUM_EXPERTS), lambda i: (i, 0)),
        out_shape=jax.ShapeDtypeStruct((TOKENS, NUM_EXPERTS), jnp.float32),
        compiler_params=pltpu.CompilerParams(
            dimension_semantics=("arbitrary",)),
    )(hidden_states, gate_weight)


def _merge_desc(ak, av, bk, bv):
    rbk = lax.rev(bk, (0,))
    rbv = lax.rev(bv, (0,))
    take_a = (ak > rbk) | ((ak == rbk) & (av < rbv))
    mk = jnp.where(take_a, ak, rbk)
    mv = jnp.where(take_a, av, rbv)
    return plsc.sort_key_val(mk, mv, descending=True)


def _sc_route(logits):
    mesh = plsc.VectorSubcoreMesh(core_axis_name="c", subcore_axis_name="s")

    @functools.partial(
        pl.kernel,
        mesh=mesh,
        out_type=[
            jax.ShapeDtypeStruct((TOKENS * TOP_K,), jnp.int32),
            jax.ShapeDtypeStruct((TOKENS * TOP_K,), jnp.float32),
        ],
        scratch_types=[
            pltpu.VMEM((R_PER_W, NUM_EXPERTS), jnp.float32),
            pltpu.VMEM((R_PER_W * TOP_K,), jnp.int32),
            pltpu.VMEM((R_PER_W * TOP_K,), jnp.float32),
        ],
        compiler_params=pltpu.CompilerParams(needs_layout_passes=False),
    )
    def sc_topk(logits_hbm, ids_hbm, wts_hbm, lg_v, ids_v, wts_v):
        wid = lax.axis_index("s") * NC + lax.axis_index("c")
        base = wid * R_PER_W
        pltpu.sync_copy(logits_hbm.at[pl.ds(base, R_PER_W)], lg_v)

        lane = lax.iota(jnp.int32, L)
        lane8 = lane < TOP_K

        @plsc.parallel_loop(0, R_PER_W, unroll=8)
        def row_body(r):
            sorted_kv = []
            for c in range(NUM_EXPERTS // L):
                k = lg_v[r, pl.ds(c * L, L)]
                sorted_kv.append(
                    plsc.sort_key_val(k, lane + c * L, descending=True))
            t01 = _merge_desc(*sorted_kv[0], *sorted_kv[1])
            t23 = _merge_desc(*sorted_kv[2], *sorted_kv[3])
            fk, fv = _merge_desc(*t01, *t23)
            e = jnp.exp(fk - jnp.max(fk))
            e8 = jnp.where(lane8, e, 0.0)
            w = e8 / jnp.sum(e8)
            pos = r * TOP_K + lane
            plsc.store_scatter(ids_v, [pos], fv, mask=lane8)
            plsc.store_scatter(wts_v, [pos], w, mask=lane8)
        pltpu.sync_copy(ids_v, ids_hbm.at[pl.ds(base * TOP_K, R_PER_W * TOP_K)])
        pltpu.sync_copy(wts_v, wts_hbm.at[pl.ds(base * TOP_K, R_PER_W * TOP_K)])

    ids_flat, wts_flat = sc_topk(logits)
    return (ids_flat.reshape(TOKENS, TOP_K), wts_flat.reshape(TOKENS, TOP_K))


@jax.jit
def kernel(hidden_states, gate_weight):
    logits = _tc_logits(hidden_states, gate_weight)
    ids, wts = _sc_route(logits)
    return ids, wts, logits
